# baseline (speedup 1.0000x reference)
import functools

import jax
import jax.numpy as jnp
from jax import lax
from jax.experimental import pallas as pl
from jax.experimental.pallas import tpu as pltpu
from jax.experimental.pallas import tpu_sc as plsc

_B = 16384
_D = 1024
_V = 1000

_info = plsc.get_sparse_core_info()
_NC = _info.num_cores
_NS = _info.num_subcores
_NW = _NC * _NS
_BPW = _B // _NW
_C = 32
_NCHUNK = _BPW // _C

_mesh = plsc.VectorSubcoreMesh(core_axis_name="c", subcore_axis_name="s")


@functools.partial(
    pl.kernel,
    mesh=_mesh,
    out_type=jax.ShapeDtypeStruct((_B, _D), jnp.float32),
    scratch_types=[
        pltpu.VMEM((_BPW,), jnp.int32),
        pltpu.VMEM((_C, _D), jnp.float32),
        pltpu.VMEM((_C, _D), jnp.float32),
        pltpu.VMEM((_C, _D), jnp.float32),
        pltpu.SemaphoreType.DMA,
        pltpu.SemaphoreType.DMA,
        pltpu.SemaphoreType.DMA,
        pltpu.SemaphoreType.DMA,
        pltpu.SemaphoreType.DMA,
        pltpu.SemaphoreType.DMA,
    ],
)
def _embed_sc(labels_hbm, table_hbm, out_hbm, idx_v, rows0, rows1, rows2,
              gsem0, gsem1, gsem2, wsem0, wsem1, wsem2):
    wid = lax.axis_index("s") * _NC + lax.axis_index("c")
    base = wid * _BPW
    rows = (rows0, rows1, rows2)
    gsem = (gsem0, gsem1, gsem2)
    wsem = (wsem0, wsem1, wsem2)
    pltpu.sync_copy(labels_hbm.at[pl.ds(base, _BPW)], idx_v)
    gathers = [None] * _NCHUNK
    writes = [None] * _NCHUNK
    for g in range(_NCHUNK):
        b = g % 3
        if g >= 3:
            writes[g - 3].wait()
        gathers[g] = pltpu.async_copy(
            table_hbm.at[idx_v.at[pl.ds(g * _C, _C)]], rows[b], gsem[b])
        if g >= 1:
            gathers[g - 1].wait()
            writes[g - 1] = pltpu.async_copy(
                rows[(g - 1) % 3],
                out_hbm.at[pl.ds(base + (g - 1) * _C, _C)],
                wsem[(g - 1) % 3])
    g = _NCHUNK - 1
    gathers[g].wait()
    writes[g] = pltpu.async_copy(
        rows[g % 3], out_hbm.at[pl.ds(base + g * _C, _C)], wsem[g % 3])
    for t in range(_NCHUNK - 3, _NCHUNK):
        writes[t].wait()


def kernel(labels, table):
    return _embed_sc(labels.astype(jnp.int32), table)


# speedup vs baseline: 1.0054x; 1.0054x over previous
import functools

import jax
import jax.numpy as jnp
from jax import lax
from jax.experimental import pallas as pl
from jax.experimental.pallas import tpu as pltpu
from jax.experimental.pallas import tpu_sc as plsc

_B = 16384
_D = 1024
_V = 1000

_info = plsc.get_sparse_core_info()
_NC = _info.num_cores
_NS = _info.num_subcores
_NW = _NC * _NS
_BPW = _B // _NW
_C = 112
_CHUNKS = (112, 112, 112, 112, 64)

_mesh = plsc.VectorSubcoreMesh(core_axis_name="c", subcore_axis_name="s")


@functools.partial(
    pl.kernel,
    mesh=_mesh,
    out_type=jax.ShapeDtypeStruct((_B, _D), jnp.float32),
    scratch_types=[
        pltpu.VMEM((_BPW,), jnp.int32),
        pltpu.VMEM((_C, _D), jnp.float32),
        pltpu.SemaphoreType.DMA,
        pltpu.SemaphoreType.DMA,
    ],
)
def _embed_sc(labels_hbm, table_hbm, out_hbm, idx_v, rows0, gsem0, wsem0):
    wid = lax.axis_index("s") * _NC + lax.axis_index("c")
    base = wid * _BPW
    pltpu.sync_copy(labels_hbm.at[pl.ds(base, _BPW)], idx_v)
    off = 0
    for n in _CHUNKS:
        pltpu.async_copy(
            table_hbm.at[idx_v.at[pl.ds(off, n)]],
            rows0.at[pl.ds(0, n)], gsem0).wait()
        pltpu.async_copy(
            rows0.at[pl.ds(0, n)],
            out_hbm.at[pl.ds(base + off, n)], wsem0).wait()
        off += n


def kernel(labels, table):
    return _embed_sc(labels.astype(jnp.int32), table)
